# baseline (speedup 1.0000x reference)
import functools

import jax
import jax.numpy as jnp
from jax import lax
from jax.experimental import pallas as pl
from jax.experimental.pallas import tpu as pltpu
from jax.experimental.pallas import tpu_sc as plsc

D_MODEL = 768
MAX_POSITION = 2048
LANES = 16
NUM_CORES = 2
NUM_SUBCORES = 16
NUM_WORKERS = NUM_CORES * NUM_SUBCORES
CHUNK = 64


def _positional_encoding(length, depth_full):
    depth = depth_full // 2
    positions = jnp.arange(0, length, dtype=jnp.float32)[:, None]
    depths = jnp.arange(depth, dtype=jnp.float32)[None, :] / depth
    angle_rates = 1.0 / (10000.0 ** depths)
    angle_rads = positions * angle_rates
    return jnp.concatenate([jnp.sin(angle_rads), jnp.cos(angle_rads)], axis=-1)


def _embed_sc(table, idx, pos, n_rows, seq_len):
    rows_per_worker = n_rows // NUM_WORKERS
    scale = jnp.float32(jnp.sqrt(jnp.float32(D_MODEL)))
    mesh = plsc.VectorSubcoreMesh(core_axis_name="c", subcore_axis_name="s")

    @functools.partial(
        pl.kernel,
        mesh=mesh,
        out_type=jax.ShapeDtypeStruct((n_rows, D_MODEL), jnp.float32),
        scratch_types=[
            pltpu.VMEM((rows_per_worker,), jnp.int32),
            pltpu.VMEM((CHUNK, D_MODEL), jnp.float32),
            pltpu.VMEM((CHUNK, D_MODEL), jnp.float32),
            pltpu.SemaphoreType.DMA,
        ],
    )
    def k(table_hbm, idx_hbm, pos_hbm, out_hbm, idx_v, rows_v, pos_v, sem):
        wid = lax.axis_index("s") * NUM_CORES + lax.axis_index("c")
        base = wid * rows_per_worker
        pbase = lax.rem(base, seq_len)
        pltpu.sync_copy(idx_hbm.at[pl.ds(base, rows_per_worker)], idx_v)

        @pl.loop(0, rows_per_worker, step=CHUNK)
        def _chunk(c):
            pltpu.async_copy(
                table_hbm.at[idx_v.at[pl.ds(c, CHUNK)]], rows_v, sem
            ).wait()
            pltpu.sync_copy(pos_hbm.at[pl.ds(pbase + c, CHUNK)], pos_v)

            @pl.loop(0, CHUNK)
            def _row(r):
                @pl.loop(0, D_MODEL, step=LANES)
                def _vec(j):
                    rows_v[r, pl.ds(j, LANES)] = (
                        rows_v[r, pl.ds(j, LANES)] * scale
                        + pos_v[r, pl.ds(j, LANES)]
                    )

            pltpu.sync_copy(rows_v, out_hbm.at[pl.ds(base + c, CHUNK)])

    return k(table, idx, pos)


def kernel(inputs, table):
    batch, seq_len = inputs.shape
    n_rows = batch * seq_len
    idx = jnp.reshape(inputs.astype(jnp.int32), (n_rows,))
    pos = _positional_encoding(MAX_POSITION, D_MODEL)[:seq_len]
    out = _embed_sc(table, idx, pos, n_rows, seq_len)
    return jnp.reshape(out, (batch, seq_len, D_MODEL))


# speedup vs baseline: 1.4252x; 1.4252x over previous
import functools

import jax
import jax.numpy as jnp
import numpy as np
from jax import lax
from jax.experimental import pallas as pl
from jax.experimental.pallas import tpu as pltpu
from jax.experimental.pallas import tpu_sc as plsc

D_MODEL = 768
MAX_POSITION = 2048
NUM_CORES = 2
NUM_SUBCORES = 16
NUM_WORKERS = NUM_CORES * NUM_SUBCORES


def _positional_encoding(length, depth_full):
    depth = depth_full // 2
    positions = jnp.arange(0, length, dtype=jnp.float32)[:, None]
    depths = jnp.arange(depth, dtype=jnp.float32)[None, :] / depth
    angle_rates = 1.0 / (10000.0 ** depths)
    angle_rads = positions * angle_rates
    return jnp.concatenate([jnp.sin(angle_rads), jnp.cos(angle_rads)], axis=-1)


def _sc_gather(table, idx):
    n = idx.shape[0]
    rpw = n // NUM_WORKERS
    mesh = plsc.VectorSubcoreMesh(core_axis_name="c", subcore_axis_name="s")

    @functools.partial(
        pl.kernel,
        mesh=mesh,
        out_type=jax.ShapeDtypeStruct((n, D_MODEL), jnp.float32),
        scratch_types=[
            pltpu.VMEM((rpw,), jnp.int32),
            pltpu.VMEM((rpw, D_MODEL), jnp.float32),
            pltpu.SemaphoreType.DMA,
        ],
    )
    def k(table_hbm, idx_hbm, out_hbm, idx_v, rows_v, sem):
        wid = lax.axis_index("s") * NUM_CORES + lax.axis_index("c")
        base = wid * rpw
        pltpu.sync_copy(idx_hbm.at[pl.ds(base, rpw)], idx_v)
        pltpu.async_copy(table_hbm.at[idx_v], rows_v, sem).wait()
        pltpu.sync_copy(rows_v, out_hbm.at[pl.ds(base, rpw)])

    return k(table, idx)


def _tc_fixup(gathered, pos, scale):
    n = gathered.shape[0]
    block = 256
    scale = float(scale)

    def body(g_ref, p_ref, o_ref):
        o_ref[...] = g_ref[...] * scale + p_ref[...]

    return pl.pallas_call(
        body,
        out_shape=jax.ShapeDtypeStruct((n, D_MODEL), jnp.float32),
        grid=(n // block,),
        in_specs=[
            pl.BlockSpec((block, D_MODEL), lambda i: (i, 0)),
            pl.BlockSpec((block, D_MODEL), lambda i: (i, 0)),
        ],
        out_specs=pl.BlockSpec((block, D_MODEL), lambda i: (i, 0)),
    )(gathered, pos)


def kernel(inputs, table):
    batch, seq_len = inputs.shape
    idx = inputs.astype(jnp.int32)
    pos = _positional_encoding(MAX_POSITION, D_MODEL)[:seq_len]
    scale = float(np.sqrt(np.float32(D_MODEL)))
    outs = []
    for b in range(batch):
        gathered = _sc_gather(table, idx[b])
        outs.append(_tc_fixup(gathered, pos, scale))
    return jnp.stack(outs, axis=0)


# speedup vs baseline: 1.8561x; 1.3024x over previous
import functools

import jax
import jax.numpy as jnp
import numpy as np
from jax import lax
from jax.experimental import pallas as pl
from jax.experimental.pallas import tpu as pltpu
from jax.experimental.pallas import tpu_sc as plsc

D_MODEL = 768
MAX_POSITION = 2048
NUM_CORES = 2
NUM_SUBCORES = 16
NUM_WORKERS = NUM_CORES * NUM_SUBCORES
CHUNK = 64


def _positional_encoding(length, depth_full):
    depth = depth_full // 2
    positions = jnp.arange(0, length, dtype=jnp.float32)[:, None]
    depths = jnp.arange(depth, dtype=jnp.float32)[None, :] / depth
    angle_rates = 1.0 / (10000.0 ** depths)
    angle_rads = positions * angle_rates
    return jnp.concatenate([jnp.sin(angle_rads), jnp.cos(angle_rads)], axis=-1)


def _sc_gather(table, idx):
    n = idx.shape[0]
    rpw = n // NUM_WORKERS
    n_chunks = rpw // CHUNK
    mesh = plsc.VectorSubcoreMesh(core_axis_name="c", subcore_axis_name="s")

    @functools.partial(
        pl.kernel,
        mesh=mesh,
        out_type=jax.ShapeDtypeStruct((n, D_MODEL), jnp.float32),
        scratch_types=[
            pltpu.VMEM((rpw,), jnp.int32),
            pltpu.VMEM((CHUNK, D_MODEL), jnp.float32),
            pltpu.VMEM((CHUNK, D_MODEL), jnp.float32),
            pltpu.SemaphoreType.DMA,
            pltpu.SemaphoreType.DMA,
        ],
    )
    def k(table_hbm, idx_hbm, out_hbm, idx_v, buf0, buf1, sem0, sem1):
        wid = lax.axis_index("s") * NUM_CORES + lax.axis_index("c")
        base = wid * rpw
        pltpu.sync_copy(idx_hbm.at[pl.ds(base, rpw)], idx_v)
        bufs = (buf0, buf1)
        sems = (sem0, sem1)
        copies = [None] * n_chunks
        copies[0] = pltpu.async_copy(
            table_hbm.at[idx_v.at[pl.ds(0, CHUNK)]], bufs[0], sems[0]
        )
        for c in range(n_chunks):
            copies[c].wait()
            if c + 1 < n_chunks:
                copies[c + 1] = pltpu.async_copy(
                    table_hbm.at[idx_v.at[pl.ds((c + 1) * CHUNK, CHUNK)]],
                    bufs[(c + 1) % 2],
                    sems[(c + 1) % 2],
                )
            pltpu.sync_copy(bufs[c % 2], out_hbm.at[pl.ds(base + c * CHUNK, CHUNK)])

    return k(table, idx)


def _tc_fixup(gathered, pos, batch, seq_len, scale):
    block = 256
    n_seq_blocks = seq_len // block

    def body(g_ref, p_ref, o_ref):
        o_ref[...] = g_ref[...] * scale + p_ref[...]

    return pl.pallas_call(
        body,
        out_shape=jax.ShapeDtypeStruct((batch * seq_len, D_MODEL), jnp.float32),
        grid=(n_seq_blocks, batch),
        in_specs=[
            pl.BlockSpec(
                (block, D_MODEL), lambda i, b: (b * n_seq_blocks + i, 0)
            ),
            pl.BlockSpec((block, D_MODEL), lambda i, b: (i, 0)),
        ],
        out_specs=pl.BlockSpec(
            (block, D_MODEL), lambda i, b: (b * n_seq_blocks + i, 0)
        ),
    )(gathered, pos)


def kernel(inputs, table):
    batch, seq_len = inputs.shape
    n_rows = batch * seq_len
    idx = jnp.reshape(inputs.astype(jnp.int32), (n_rows,))
    pos = _positional_encoding(MAX_POSITION, D_MODEL)[:seq_len]
    scale = float(np.sqrt(np.float32(D_MODEL)))
    gathered = _sc_gather(table, idx)
    out = _tc_fixup(gathered, pos, batch, seq_len, scale)
    return jnp.reshape(out, (batch, seq_len, D_MODEL))
